# baseline (speedup 1.0000x reference)
import functools

import jax
import jax.numpy as jnp
from jax import lax
from jax.experimental import pallas as pl
from jax.experimental.pallas import tpu as pltpu
from jax.experimental.pallas import tpu_sc as plsc

NNODE = 10000
NPAD = 10240
NC = 2
NS = 16
NW = NC * NS
EDGES = 160000
EPT = EDGES // NW
BB = 128
NB = 40
RZ = NPAD // NS

_mesh = plsc.VectorSubcoreMesh(
    core_axis_name="c", subcore_axis_name="s", num_cores=NC, num_subcores=NS)


@functools.partial(
    pl.kernel,
    out_type=jax.ShapeDtypeStruct((NC, NPAD, 128), jnp.float32),
    mesh=_mesh,
    scratch_types=[
        pltpu.VMEM((NB, BB), jnp.int32),
        pltpu.VMEM((BB, 128), jnp.float32),
        pltpu.VMEM_SHARED((NPAD, 128), jnp.float32),
    ],
)
def _sc_deg(dst_hbm, ones_hbm, zeros_hbm, out_hbm, idx_v, ones_v, acc):
    cid = lax.axis_index("c")
    sid = lax.axis_index("s")
    wid = sid * NC + cid

    pltpu.sync_copy(ones_hbm, ones_v)
    pltpu.sync_copy(dst_hbm.at[wid], idx_v)
    pltpu.sync_copy(zeros_hbm, acc.at[pl.ds(sid * RZ, RZ)])
    plsc.subcore_barrier()

    def body(j, carry):
        pltpu.sync_copy(ones_v, acc.at[idx_v.at[j]], add=True)
        return carry
    lax.fori_loop(0, NB, body, 0)
    plsc.subcore_barrier()

    pltpu.sync_copy(acc.at[pl.ds(sid * RZ, RZ)],
                    out_hbm.at[cid, pl.ds(sid * RZ, RZ)])


def _make_prop(nc_chunks):
    @functools.partial(
        pl.kernel,
        out_type=jax.ShapeDtypeStruct((NC, nc_chunks, NPAD, 128), jnp.float32),
        mesh=_mesh,
        scratch_types=[
            pltpu.VMEM((NB, BB), jnp.int32),
            pltpu.VMEM((NB, BB), jnp.int32),
            pltpu.VMEM((2, BB, 128), jnp.float32),
            pltpu.VMEM_SHARED((NPAD, 128), jnp.float32),
            pltpu.SemaphoreType.DMA,
            pltpu.SemaphoreType.DMA,
            pltpu.SemaphoreType.DMA,
            pltpu.SemaphoreType.DMA,
        ],
    )
    def _prop(*refs):
        g_refs = refs[:nc_chunks]
        (src_hbm, dst_hbm, zeros_hbm, out_hbm,
         src_v, dst_v, gbuf, acc) = refs[nc_chunks:nc_chunks + 8]
        gsem = refs[nc_chunks + 8:nc_chunks + 10]
        ssem = refs[nc_chunks + 10:nc_chunks + 12]
        cid = lax.axis_index("c")
        sid = lax.axis_index("s")
        wid = sid * NC + cid

        pltpu.sync_copy(src_hbm.at[wid], src_v)
        pltpu.sync_copy(dst_hbm.at[wid], dst_v)

        def fire_gather(g, j, b):
            pltpu.async_copy(g.at[src_v.at[j, pl.ds(0, BB // 2)]],
                             gbuf.at[b, pl.ds(0, BB // 2)], gsem[b])
            pltpu.async_copy(g.at[src_v.at[j, pl.ds(BB // 2, BB // 2)]],
                             gbuf.at[b, pl.ds(BB // 2, BB // 2)], gsem[b])

        def wait_gather(b):
            pltpu.make_async_copy(zeros_hbm.at[pl.ds(0, BB)],
                                  gbuf.at[b], gsem[b]).wait()

        def fire_scatter(j, b):
            pltpu.async_copy(gbuf.at[b], acc.at[dst_v.at[j]], ssem[b],
                             add=True)

        def wait_scatter(b):
            pltpu.make_async_copy(zeros_hbm.at[pl.ds(0, BB)],
                                  gbuf.at[b], ssem[b]).wait()

        for c in range(nc_chunks):
            pltpu.sync_copy(zeros_hbm, acc.at[pl.ds(sid * RZ, RZ)])
            plsc.subcore_barrier()

            g = g_refs[c]
            fire_gather(g, 0, 0)
            fire_gather(g, 1, 1)

            def group(gi, carry):
                j0 = gi * 2
                for b in range(2):
                    wait_gather(b)
                    fire_scatter(j0 + b, b)
                    wait_scatter(b)
                    fire_gather(g, j0 + b + 2, b)
                return carry
            lax.fori_loop(0, (NB - 2) // 2, group, 0)

            for j in range(NB - 2, NB):
                b = j % 2
                wait_gather(b)
                fire_scatter(j, b)
                wait_scatter(b)
            plsc.subcore_barrier()

            pltpu.sync_copy(acc.at[pl.ds(sid * RZ, RZ)],
                            out_hbm.at[cid, c, pl.ds(sid * RZ, RZ)])
            plsc.subcore_barrier()

    return _prop


_sc_prop2 = _make_prop(2)
_sc_prop4 = _make_prop(4)


R = 1000
GRID = NNODE // R


def _dinv_of(degp):
    deg = degp[0, :, :1] + degp[1, :, :1] + 1.0
    return lax.rsqrt(deg)


def _mm(a, b):
    return lax.dot_general(a, b, (((1,), (0,)), ((), ())),
                           precision=lax.Precision.HIGHEST,
                           preferred_element_type=jnp.float32)


def _tc_prep_body(degp_ref, x_ref, ga_ref, gb_ref):
    dinv = _dinv_of(degp_ref[...])
    g = x_ref[...] * dinv
    ga_ref[...] = g[:, :128]
    gb_ref[...] = g[:, 128:]


_tc_prep = pl.pallas_call(
    _tc_prep_body,
    grid=(GRID,),
    in_specs=[
        pl.BlockSpec((NC, R, 128), lambda i: (0, i, 0)),
        pl.BlockSpec((R, 256), lambda i: (i, 0)),
    ],
    out_specs=[pl.BlockSpec((R, 128), lambda i: (i, 0))] * 2,
    out_shape=[jax.ShapeDtypeStruct((NNODE, 128), jnp.float32)] * 2,
)


def _tc_layer1_body(degp_ref, x_ref, s0_ref, W1_ref, b1_ref,
                    h1_ref, g0_ref, g1_ref, g2_ref, g3_ref):
    dinv = _dinv_of(degp_ref[...])
    s0 = s0_ref[...]
    ssum = s0[0] + s0[1]
    scat = jnp.concatenate([ssum[0], ssum[1]], axis=-1)
    xp = dinv * scat + (dinv * dinv) * x_ref[...]
    h1 = jnp.maximum(_mm(xp, W1_ref[...]) + b1_ref[...], 0.0)
    h1_ref[...] = h1
    g = h1 * dinv
    g0_ref[...] = g[:, 0:128]
    g1_ref[...] = g[:, 128:256]
    g2_ref[...] = g[:, 256:384]
    g3_ref[...] = g[:, 384:512]


_tc_layer1 = pl.pallas_call(
    _tc_layer1_body,
    grid=(GRID,),
    in_specs=[
        pl.BlockSpec((NC, R, 128), lambda i: (0, i, 0)),
        pl.BlockSpec((R, 256), lambda i: (i, 0)),
        pl.BlockSpec((NC, 2, R, 128), lambda i: (0, 0, i, 0)),
        pl.BlockSpec((256, 512), lambda i: (0, 0)),
        pl.BlockSpec((1, 512), lambda i: (0, 0)),
    ],
    out_specs=[pl.BlockSpec((R, 512), lambda i: (i, 0))]
    + [pl.BlockSpec((R, 128), lambda i: (i, 0))] * 4,
    out_shape=[jax.ShapeDtypeStruct((NNODE, 512), jnp.float32)]
    + [jax.ShapeDtypeStruct((NNODE, 128), jnp.float32)] * 4,
)


def _tc_final_body(degp_ref, h1_ref, s1_ref, W2_ref, b2_ref,
                   Wa_ref, Wb_ref, bl_ref, out_ref):
    dinv = _dinv_of(degp_ref[...])
    s1 = s1_ref[...]
    ss = s1[0] + s1[1]
    hcat = jnp.concatenate([ss[0], ss[1], ss[2], ss[3]], axis=-1)
    h1 = h1_ref[...]
    h1p = dinv * hcat + (dinv * dinv) * h1
    h2 = jnp.maximum(_mm(h1p, W2_ref[...]) + b2_ref[...], 0.0)
    logits = _mm(h1, Wa_ref[...]) + _mm(h2, Wb_ref[...]) + bl_ref[...]
    m = jnp.max(logits, axis=1, keepdims=True)
    e = jnp.exp(logits - m)
    out_ref[...] = (logits - m) - jnp.log(jnp.sum(e, axis=1, keepdims=True))


_tc_final = pl.pallas_call(
    _tc_final_body,
    grid=(GRID,),
    in_specs=[
        pl.BlockSpec((NC, R, 128), lambda i: (0, i, 0)),
        pl.BlockSpec((R, 512), lambda i: (i, 0)),
        pl.BlockSpec((NC, 4, R, 128), lambda i: (0, 0, i, 0)),
        pl.BlockSpec((512, 512), lambda i: (0, 0)),
        pl.BlockSpec((1, 512), lambda i: (0, 0)),
        pl.BlockSpec((512, 64), lambda i: (0, 0)),
        pl.BlockSpec((512, 64), lambda i: (0, 0)),
        pl.BlockSpec((1, 64), lambda i: (0, 0)),
    ],
    out_specs=pl.BlockSpec((R, 64), lambda i: (i, 0)),
    out_shape=jax.ShapeDtypeStruct((NNODE, 64), jnp.float32),
)


@jax.jit
def kernel(x, edge_index, W1, b1, W2, b2, Wlin, blin):
    src = edge_index[0].reshape(NW, EPT)
    dst = edge_index[1].reshape(NW, EPT)
    pad = NB * BB - EPT
    src3 = jnp.pad(src, ((0, 0), (0, pad))).reshape(NW, NB, BB)
    dst3 = jnp.pad(dst, ((0, 0), (0, pad)),
                   constant_values=NNODE).reshape(NW, NB, BB)

    onesb = jnp.ones((BB, 128), jnp.float32)
    zerosb = jnp.zeros((RZ, 128), jnp.float32)
    degp = _sc_deg(dst3, onesb, zerosb)
    g0a, g0b = _tc_prep(degp, x)
    s0 = _sc_prop2(g0a, g0b, src3, dst3, zerosb)
    h1, g1a, g1b, g1c, g1d = _tc_layer1(
        degp, x, s0, W1, b1.reshape(1, 512))
    s1 = _sc_prop4(g1a, g1b, g1c, g1d, src3, dst3, zerosb)
    out = _tc_final(degp, h1, s1, W2, b2.reshape(1, 512),
                    Wlin[:512], Wlin[512:], blin.reshape(1, 64))
    return out


# speedup vs baseline: 1.0546x; 1.0546x over previous
import functools

import jax
import jax.numpy as jnp
from jax import lax
from jax.experimental import pallas as pl
from jax.experimental.pallas import tpu as pltpu
from jax.experimental.pallas import tpu_sc as plsc

NNODE = 10000
NPAD = 10240
NC = 2
NS = 16
NW = NC * NS
EDGES = 160000
EPT = EDGES // NW
BB = 128
NB = 40
RZ = NPAD // NS

_mesh = plsc.VectorSubcoreMesh(
    core_axis_name="c", subcore_axis_name="s", num_cores=NC, num_subcores=NS)


@functools.partial(
    pl.kernel,
    out_type=jax.ShapeDtypeStruct((NC, NPAD, 128), jnp.float32),
    mesh=_mesh,
    scratch_types=[
        pltpu.VMEM((NB, BB), jnp.int32),
        pltpu.VMEM((BB, 128), jnp.float32),
        pltpu.VMEM_SHARED((NPAD, 128), jnp.float32),
    ],
)
def _sc_deg(dst_hbm, ones_hbm, zeros_hbm, out_hbm, idx_v, ones_v, acc):
    cid = lax.axis_index("c")
    sid = lax.axis_index("s")
    wid = sid * NC + cid

    pltpu.sync_copy(ones_hbm, ones_v)
    pltpu.sync_copy(dst_hbm.at[wid], idx_v)
    pltpu.sync_copy(zeros_hbm, acc.at[pl.ds(sid * RZ, RZ)])
    plsc.subcore_barrier()

    def body(j, carry):
        pltpu.sync_copy(ones_v, acc.at[idx_v.at[j]], add=True)
        return carry
    lax.fori_loop(0, NB, body, 0)
    plsc.subcore_barrier()

    pltpu.sync_copy(acc.at[pl.ds(sid * RZ, RZ)],
                    out_hbm.at[cid, pl.ds(sid * RZ, RZ)])


def _make_prop(nc_chunks):
    @functools.partial(
        pl.kernel,
        out_type=jax.ShapeDtypeStruct((NC, nc_chunks, NPAD, 128), jnp.float32),
        mesh=_mesh,
        scratch_types=[
            pltpu.VMEM((NB, BB), jnp.int32),
            pltpu.VMEM((NB, BB), jnp.int32),
            pltpu.VMEM((2, BB, 128), jnp.float32),
            pltpu.VMEM_SHARED((NPAD, 128), jnp.float32),
            pltpu.SemaphoreType.DMA,
            pltpu.SemaphoreType.DMA,
            pltpu.SemaphoreType.DMA,
            pltpu.SemaphoreType.DMA,
        ],
    )
    def _prop(*refs):
        g_refs = refs[:nc_chunks]
        (src_hbm, dst_hbm, zeros_hbm, out_hbm,
         src_v, dst_v, gbuf, acc) = refs[nc_chunks:nc_chunks + 8]
        gsem = refs[nc_chunks + 8:nc_chunks + 10]
        ssem = refs[nc_chunks + 10:nc_chunks + 12]
        cid = lax.axis_index("c")
        sid = lax.axis_index("s")
        wid = sid * NC + cid

        pltpu.sync_copy(src_hbm.at[wid], src_v)
        pltpu.sync_copy(dst_hbm.at[wid], dst_v)

        def fire_gather(g, j, b):
            pltpu.async_copy(g.at[src_v.at[j, pl.ds(0, BB // 2)]],
                             gbuf.at[b, pl.ds(0, BB // 2)], gsem[b])
            pltpu.async_copy(g.at[src_v.at[j, pl.ds(BB // 2, BB // 2)]],
                             gbuf.at[b, pl.ds(BB // 2, BB // 2)], gsem[b])

        def wait_gather(b):
            pltpu.make_async_copy(zeros_hbm.at[pl.ds(0, BB)],
                                  gbuf.at[b], gsem[b]).wait()

        def fire_scatter(j, b):
            pltpu.async_copy(gbuf.at[b], acc.at[dst_v.at[j]], ssem[b],
                             add=True)

        def wait_scatter(b):
            pltpu.make_async_copy(zeros_hbm.at[pl.ds(0, BB)],
                                  gbuf.at[b], ssem[b]).wait()

        pltpu.sync_copy(zeros_hbm, acc.at[pl.ds(sid * RZ, RZ)])
        fire_gather(g_refs[0], 0, 0)
        fire_gather(g_refs[0], 1, 1)
        plsc.subcore_barrier()

        for c in range(nc_chunks):
            g = g_refs[c]

            def group(gi, carry):
                j0 = gi * 2
                for b in range(2):
                    wait_gather(b)
                    fire_scatter(j0 + b, b)
                    wait_scatter(b)
                    fire_gather(g, j0 + b + 2, b)
                return carry
            lax.fori_loop(0, (NB - 2) // 2, group, 0)

            for j in range(NB - 2, NB):
                b = j % 2
                wait_gather(b)
                fire_scatter(j, b)
                wait_scatter(b)
            if c + 1 < nc_chunks:
                fire_gather(g_refs[c + 1], 0, 0)
                fire_gather(g_refs[c + 1], 1, 1)
            plsc.subcore_barrier()

            pltpu.sync_copy(acc.at[pl.ds(sid * RZ, RZ)],
                            out_hbm.at[cid, c, pl.ds(sid * RZ, RZ)])
            if c + 1 < nc_chunks:
                pltpu.sync_copy(zeros_hbm, acc.at[pl.ds(sid * RZ, RZ)])
                plsc.subcore_barrier()

    return _prop


_sc_prop2 = _make_prop(2)
_sc_prop4 = _make_prop(4)


R = 1000
GRID = NNODE // R


def _dinv_of(degp):
    deg = degp[0, :, :1] + degp[1, :, :1] + 1.0
    return lax.rsqrt(deg)


def _mm(a, b):
    return lax.dot_general(a, b, (((1,), (0,)), ((), ())),
                           preferred_element_type=jnp.float32)


def _tc_prep_body(degp_ref, x_ref, ga_ref, gb_ref):
    dinv = _dinv_of(degp_ref[...])
    g = x_ref[...] * dinv
    ga_ref[...] = g[:, :128]
    gb_ref[...] = g[:, 128:]


_tc_prep = pl.pallas_call(
    _tc_prep_body,
    grid=(GRID,),
    in_specs=[
        pl.BlockSpec((NC, R, 128), lambda i: (0, i, 0)),
        pl.BlockSpec((R, 256), lambda i: (i, 0)),
    ],
    out_specs=[pl.BlockSpec((R, 128), lambda i: (i, 0))] * 2,
    out_shape=[jax.ShapeDtypeStruct((NNODE, 128), jnp.float32)] * 2,
)


def _tc_layer1_body(degp_ref, x_ref, s0_ref, W1_ref, b1_ref,
                    h1_ref, g0_ref, g1_ref, g2_ref, g3_ref):
    dinv = _dinv_of(degp_ref[...])
    s0 = s0_ref[...]
    ssum = s0[0] + s0[1]
    scat = jnp.concatenate([ssum[0], ssum[1]], axis=-1)
    xp = dinv * scat + (dinv * dinv) * x_ref[...]
    h1 = jnp.maximum(_mm(xp, W1_ref[...]) + b1_ref[...], 0.0)
    h1_ref[...] = h1
    g = h1 * dinv
    g0_ref[...] = g[:, 0:128]
    g1_ref[...] = g[:, 128:256]
    g2_ref[...] = g[:, 256:384]
    g3_ref[...] = g[:, 384:512]


_tc_layer1 = pl.pallas_call(
    _tc_layer1_body,
    grid=(GRID,),
    in_specs=[
        pl.BlockSpec((NC, R, 128), lambda i: (0, i, 0)),
        pl.BlockSpec((R, 256), lambda i: (i, 0)),
        pl.BlockSpec((NC, 2, R, 128), lambda i: (0, 0, i, 0)),
        pl.BlockSpec((256, 512), lambda i: (0, 0)),
        pl.BlockSpec((1, 512), lambda i: (0, 0)),
    ],
    out_specs=[pl.BlockSpec((R, 512), lambda i: (i, 0))]
    + [pl.BlockSpec((R, 128), lambda i: (i, 0))] * 4,
    out_shape=[jax.ShapeDtypeStruct((NNODE, 512), jnp.float32)]
    + [jax.ShapeDtypeStruct((NNODE, 128), jnp.float32)] * 4,
)


def _tc_final_body(degp_ref, h1_ref, s1_ref, W2_ref, b2_ref,
                   Wa_ref, Wb_ref, bl_ref, out_ref):
    dinv = _dinv_of(degp_ref[...])
    s1 = s1_ref[...]
    ss = s1[0] + s1[1]
    hcat = jnp.concatenate([ss[0], ss[1], ss[2], ss[3]], axis=-1)
    h1 = h1_ref[...]
    h1p = dinv * hcat + (dinv * dinv) * h1
    h2 = jnp.maximum(_mm(h1p, W2_ref[...]) + b2_ref[...], 0.0)
    logits = _mm(h1, Wa_ref[...]) + _mm(h2, Wb_ref[...]) + bl_ref[...]
    m = jnp.max(logits, axis=1, keepdims=True)
    e = jnp.exp(logits - m)
    out_ref[...] = (logits - m) - jnp.log(jnp.sum(e, axis=1, keepdims=True))


_tc_final = pl.pallas_call(
    _tc_final_body,
    grid=(GRID,),
    in_specs=[
        pl.BlockSpec((NC, R, 128), lambda i: (0, i, 0)),
        pl.BlockSpec((R, 512), lambda i: (i, 0)),
        pl.BlockSpec((NC, 4, R, 128), lambda i: (0, 0, i, 0)),
        pl.BlockSpec((512, 512), lambda i: (0, 0)),
        pl.BlockSpec((1, 512), lambda i: (0, 0)),
        pl.BlockSpec((512, 64), lambda i: (0, 0)),
        pl.BlockSpec((512, 64), lambda i: (0, 0)),
        pl.BlockSpec((1, 64), lambda i: (0, 0)),
    ],
    out_specs=pl.BlockSpec((R, 64), lambda i: (i, 0)),
    out_shape=jax.ShapeDtypeStruct((NNODE, 64), jnp.float32),
)


@jax.jit
def kernel(x, edge_index, W1, b1, W2, b2, Wlin, blin):
    src = edge_index[0].reshape(NW, EPT)
    dst = edge_index[1].reshape(NW, EPT)
    pad = NB * BB - EPT
    src3 = jnp.pad(src, ((0, 0), (0, pad))).reshape(NW, NB, BB)
    dst3 = jnp.pad(dst, ((0, 0), (0, pad)),
                   constant_values=NNODE).reshape(NW, NB, BB)

    onesb = jnp.ones((BB, 128), jnp.float32)
    zerosb = jnp.zeros((RZ, 128), jnp.float32)
    degp = _sc_deg(dst3, onesb, zerosb)
    g0a, g0b = _tc_prep(degp, x)
    s0 = _sc_prop2(g0a, g0b, src3, dst3, zerosb)
    h1, g1a, g1b, g1c, g1d = _tc_layer1(
        degp, x, s0, W1, b1.reshape(1, 512))
    s1 = _sc_prop4(g1a, g1b, g1c, g1d, src3, dst3, zerosb)
    out = _tc_final(degp, h1, s1, W2, b2.reshape(1, 512),
                    Wlin[:512], Wlin[512:], blin.reshape(1, 64))
    return out
